# baseline (speedup 1.0000x reference)
import jax
import jax.numpy as jnp
from jax.experimental import pallas as pl

_B, _N, _F, _K = 32, 8192, 16, 1024
_BB = 8
_EPS = 1e-10
_IMIN = -2147483648


def _min_kernel(x_ref, o_ref):
    b = pl.program_id(0)
    m = jnp.full((1, 1), jnp.min(jnp.abs(x_ref[...])), jnp.float32)

    @pl.when(b == 0)
    def _():
        o_ref[:, :] = m

    @pl.when(b != 0)
    def _():
        o_ref[:, :] = jnp.minimum(o_ref[:, :], m)


def _select_kernel(xt_ref, m0t_ref, minv_ref, out_ref, mask_ref):
    x = xt_ref[...]
    m0 = m0t_ref[...]
    v = (x + (minv_ref[:, :] + _EPS)[:, :, None]) * m0
    bits = jax.lax.bitcast_convert_type(v, jnp.int32)
    keys = jnp.where(bits < 0, bits ^ jnp.int32(0x7FFFFFFF), bits)

    cnt0 = jnp.sum((keys >= 0).astype(jnp.int32), axis=2, keepdims=True)
    acc0 = cnt0 >= _K
    p0 = jnp.where(acc0, jnp.int32(_IMIN), jnp.int32(0))
    c0 = jnp.where(acc0, cnt0, jnp.int32(_N))

    vb = jax.lax.bitcast_convert_type(
        (bits >> 16).astype(jnp.int16), jnp.bfloat16)

    def hcond(ipc):
        i, _, c = ipc
        return jnp.logical_and(i < 16, jnp.logical_not(jnp.all(c == _K)))

    def hstep(i, p, c):
        cand = p | jax.lax.shift_left(jnp.int32(1),
                                      jnp.maximum(31 - i, jnp.int32(16)))
        ck = cand ^ jnp.int32(_IMIN)
        raw = jnp.where(ck >= 0, ck, ck ^ jnp.int32(0x7FFFFFFF))
        cb = jax.lax.bitcast_convert_type((raw >> 16).astype(jnp.int16),
                                          jnp.bfloat16)
        cnt = jnp.sum((vb >= cb).astype(jnp.int32), axis=2, keepdims=True)
        acc = cnt >= _K
        return jnp.where(acc, cand, p), jnp.where(acc, cnt, c)

    def hbody(ipc):
        i, p, c = ipc
        p, c = hstep(i, p, c)
        p, c = hstep(i + 1, p, c)
        return i + 2, p, c

    _, p, c = jax.lax.while_loop(hcond, hbody, (jnp.int32(1), p0, c0))

    def vcond(ipc):
        i, _, c = ipc
        return jnp.logical_and(i < 32, jnp.logical_not(jnp.all(c == _K)))

    def vstep(i, p, c):
        cand = p | jax.lax.shift_left(jnp.int32(1),
                                      jnp.maximum(31 - i, jnp.int32(0)))
        cnt = jnp.sum((keys >= (cand ^ jnp.int32(_IMIN))).astype(jnp.int32),
                      axis=2, keepdims=True)
        acc = cnt >= _K
        return jnp.where(acc, cand, p), jnp.where(acc, cnt, c)

    def vbody(ipc):
        i, p, c = ipc
        p, c = vstep(i, p, c)
        p, c = vstep(i + 1, p, c)
        return i + 2, p, c

    _, p, c = jax.lax.while_loop(vcond, vbody, (jnp.int32(16), p, c))
    tkey = p ^ jnp.int32(_IMIN)

    no_ties = jnp.all(c == _K)

    @pl.when(no_ties)
    def _():
        sel = keys >= tkey
        maskf = jnp.max(sel.astype(jnp.float32), axis=1, keepdims=True)
        mask_ref[...] = maskf
        out_ref[...] = x * maskf

    @pl.when(jnp.logical_not(no_ties))
    def _():
        gt = keys > tkey
        eq = keys == tkey
        iota = jax.lax.broadcasted_iota(jnp.int32, (_BB, _F, _N), 2)
        iota_m = jnp.where(eq, iota, jnp.int32(_N))
        g0 = jnp.sum(gt.astype(jnp.int32), axis=2, keepdims=True)

        def ibody(i, p2):
            cand = p2 | jax.lax.shift_left(jnp.int32(1), 12 - i)
            cnt = g0 + jnp.sum((iota_m <= cand).astype(jnp.int32), axis=2,
                               keepdims=True)
            return jnp.where(cnt < _K, cand, p2)

        p2 = jax.lax.fori_loop(0, 13, ibody,
                               jnp.zeros((_BB, _F, 1), jnp.int32))
        gp = g0 + jnp.sum((iota_m <= p2).astype(jnp.int32), axis=2,
                          keepdims=True)
        jstar = p2 + (gp < _K).astype(jnp.int32)

        sel = gt | (iota_m <= jstar)
        maskf = jnp.max(sel.astype(jnp.float32), axis=1, keepdims=True)
        mask_ref[...] = maskf
        out_ref[...] = x * maskf


@jax.jit
def kernel(input, mask, init_mask):
    del mask
    xt = jnp.transpose(input, (0, 2, 1))
    m0t = jnp.transpose(init_mask, (0, 2, 1))

    minv = pl.pallas_call(
        _min_kernel,
        grid=(_B // _BB,),
        in_specs=[pl.BlockSpec((_BB, _F, _N), lambda b: (b, 0, 0))],
        out_specs=pl.BlockSpec((1, 1), lambda b: (0, 0)),
        out_shape=jax.ShapeDtypeStruct((1, 1), jnp.float32),
    )(xt)

    out_t, mask_t = pl.pallas_call(
        _select_kernel,
        grid=(_B // _BB,),
        in_specs=[
            pl.BlockSpec((_BB, _F, _N), lambda b: (b, 0, 0)),
            pl.BlockSpec((_BB, 1, _N), lambda b: (b, 0, 0)),
            pl.BlockSpec((1, 1), lambda b: (0, 0)),
        ],
        out_specs=[
            pl.BlockSpec((_BB, _F, _N), lambda b: (b, 0, 0)),
            pl.BlockSpec((_BB, 1, _N), lambda b: (b, 0, 0)),
        ],
        out_shape=[
            jax.ShapeDtypeStruct((_B, _F, _N), jnp.float32),
            jax.ShapeDtypeStruct((_B, 1, _N), jnp.float32),
        ],
    )(xt, m0t, minv)

    updated_mask = jnp.reshape(mask_t, (_B, _N, 1))
    masked_out = jnp.transpose(out_t, (0, 2, 1))
    return (updated_mask, masked_out)


# speedup vs baseline: 1.4341x; 1.4341x over previous
import jax
import jax.numpy as jnp
from jax.experimental import pallas as pl

_B, _N, _F, _K = 32, 8192, 16, 1024
_BB = 8
_EPS = 1e-10
_IMIN = -2147483648


def _min_kernel(x_ref, o_ref):
    b = pl.program_id(0)
    m = jnp.full((1, 1), jnp.min(jnp.abs(x_ref[...])), jnp.float32)

    @pl.when(b == 0)
    def _():
        o_ref[:, :] = m

    @pl.when(b != 0)
    def _():
        o_ref[:, :] = jnp.minimum(o_ref[:, :], m)


def _select_kernel(xt_ref, m0t_ref, minv_ref, out_ref, mask_ref):
    x = xt_ref[...]
    m0 = m0t_ref[...]
    v = (x + (minv_ref[:, :] + _EPS)[:, :, None]) * m0
    bits = jax.lax.bitcast_convert_type(v, jnp.int32)
    keys = jnp.where(bits < 0, bits ^ jnp.int32(0x7FFFFFFF), bits)

    cnt0 = jnp.sum((keys >= 0).astype(jnp.int32), axis=2, keepdims=True)
    acc0 = cnt0 >= _K
    p0 = jnp.where(acc0, jnp.int32(_IMIN), jnp.int32(0))
    c0 = jnp.where(acc0, cnt0, jnp.int32(_N))

    def vcond(ipc):
        i, _, c = ipc
        return jnp.logical_and(i < 32, jnp.logical_not(jnp.all(c == _K)))

    def vstep(i, p, c):
        cand = p | jax.lax.shift_left(jnp.int32(1),
                                      jnp.maximum(31 - i, jnp.int32(0)))
        cnt = jnp.sum((keys >= (cand ^ jnp.int32(_IMIN))).astype(jnp.int32),
                      axis=2, keepdims=True)
        acc = cnt >= _K
        return jnp.where(acc, cand, p), jnp.where(acc, cnt, c)

    def vbody(ipc):
        i, p, c = ipc
        p, c = vstep(i, p, c)
        p, c = vstep(i + 1, p, c)
        return i + 2, p, c

    _, p, c = jax.lax.while_loop(vcond, vbody, (jnp.int32(1), p0, c0))
    tkey = p ^ jnp.int32(_IMIN)

    no_ties = jnp.all(c == _K)

    @pl.when(no_ties)
    def _():
        sel = keys >= tkey
        maskf = jnp.max(sel.astype(jnp.float32), axis=1, keepdims=True)
        mask_ref[...] = maskf
        out_ref[...] = x * maskf

    @pl.when(jnp.logical_not(no_ties))
    def _():
        gt = keys > tkey
        eq = keys == tkey
        iota = jax.lax.broadcasted_iota(jnp.int32, (_BB, _F, _N), 2)
        iota_m = jnp.where(eq, iota, jnp.int32(_N))
        g0 = jnp.sum(gt.astype(jnp.int32), axis=2, keepdims=True)

        def ibody(i, p2):
            cand = p2 | jax.lax.shift_left(jnp.int32(1), 12 - i)
            cnt = g0 + jnp.sum((iota_m <= cand).astype(jnp.int32), axis=2,
                               keepdims=True)
            return jnp.where(cnt < _K, cand, p2)

        p2 = jax.lax.fori_loop(0, 13, ibody,
                               jnp.zeros((_BB, _F, 1), jnp.int32))
        gp = g0 + jnp.sum((iota_m <= p2).astype(jnp.int32), axis=2,
                          keepdims=True)
        jstar = p2 + (gp < _K).astype(jnp.int32)

        sel = gt | (iota_m <= jstar)
        maskf = jnp.max(sel.astype(jnp.float32), axis=1, keepdims=True)
        mask_ref[...] = maskf
        out_ref[...] = x * maskf


@jax.jit
def kernel(input, mask, init_mask):
    del mask
    xt = jnp.transpose(input, (0, 2, 1))
    m0t = jnp.transpose(init_mask, (0, 2, 1))

    minv = pl.pallas_call(
        _min_kernel,
        grid=(_B // _BB,),
        in_specs=[pl.BlockSpec((_BB, _F, _N), lambda b: (b, 0, 0))],
        out_specs=pl.BlockSpec((1, 1), lambda b: (0, 0)),
        out_shape=jax.ShapeDtypeStruct((1, 1), jnp.float32),
    )(xt)

    out_t, mask_t = pl.pallas_call(
        _select_kernel,
        grid=(_B // _BB,),
        in_specs=[
            pl.BlockSpec((_BB, _F, _N), lambda b: (b, 0, 0)),
            pl.BlockSpec((_BB, 1, _N), lambda b: (b, 0, 0)),
            pl.BlockSpec((1, 1), lambda b: (0, 0)),
        ],
        out_specs=[
            pl.BlockSpec((_BB, _F, _N), lambda b: (b, 0, 0)),
            pl.BlockSpec((_BB, 1, _N), lambda b: (b, 0, 0)),
        ],
        out_shape=[
            jax.ShapeDtypeStruct((_B, _F, _N), jnp.float32),
            jax.ShapeDtypeStruct((_B, 1, _N), jnp.float32),
        ],
    )(xt, m0t, minv)

    updated_mask = jnp.reshape(mask_t, (_B, _N, 1))
    masked_out = jnp.transpose(out_t, (0, 2, 1))
    return (updated_mask, masked_out)
